# baseline (speedup 1.0000x reference)
from functools import partial

import jax
import jax.numpy as jnp
import numpy as np
from jax.experimental import pallas as pl
from jax.experimental.pallas import tpu as pltpu
from jax.experimental.pallas import tpu_sc as plsc
from jax.experimental.shard_map import shard_map
from jax.sharding import Mesh, PartitionSpec as P

_BN = 512


def _tc_body(th_ref, wt_ref, b_ref, wv_ref, w_ref, v_ref, wr_ref):
    i = pl.program_id(0)
    bn, C = w_ref.shape
    th = th_ref[...]
    w = jax.lax.dot_general(th, wt_ref[...], (((0,), (0,)), ((), ())),
                            preferred_element_type=jnp.float32)
    w = w + b_ref[...]
    w_ref[...] = w
    v_ref[...] = jax.lax.dot_general(th, wv_ref[...], (((0,), (0,)), ((), ())),
                                     preferred_element_type=jnp.float32)
    colio = jax.lax.broadcasted_iota(jnp.int32, (bn, C), 1)
    rowmax = jnp.max(w, axis=1, keepdims=True)
    idx = jnp.min(jnp.where(w == rowmax, colio, C), axis=1, keepdims=True)
    rowio = jax.lax.broadcasted_iota(jnp.int32, (bn, C), 0) + i * bn
    blkmax = jnp.max(jnp.where(colio == idx, rowio, -1), axis=0, keepdims=True)

    @pl.when(i == 0)
    def _():
        wr_ref[...] = blkmax

    @pl.when(i > 0)
    def _():
        wr_ref[...] = jnp.maximum(wr_ref[...], blkmax)


def _tc_call(thetas, Wt, b2, Wv):
    IN, N = thetas.shape
    C = Wt.shape[1]
    L = Wv.shape[1]
    grid = (N // _BN,)
    return pl.pallas_call(
        _tc_body,
        grid=grid,
        in_specs=[
            pl.BlockSpec((IN, _BN), lambda i: (0, i)),
            pl.BlockSpec((IN, C), lambda i: (0, 0)),
            pl.BlockSpec((1, C), lambda i: (0, 0)),
            pl.BlockSpec((IN, L), lambda i: (0, 0)),
        ],
        out_specs=[
            pl.BlockSpec((_BN, C), lambda i: (i, 0)),
            pl.BlockSpec((_BN, L), lambda i: (i, 0)),
            pl.BlockSpec((1, C), lambda i: (0, 0)),
        ],
        out_shape=[
            jax.ShapeDtypeStruct((N, C), jnp.float32),
            jax.ShapeDtypeStruct((N, L), jnp.float32),
            jax.ShapeDtypeStruct((1, C), jnp.int32),
        ],
    )(thetas, Wt, b2, Wv)


def _sc_gather(src, gidx, gw):
    Cl = gidx.shape[1]
    L = src.shape[1]
    mesh = plsc.VectorSubcoreMesh(core_axis_name="c", subcore_axis_name="s")

    @pl.kernel(out_type=jax.ShapeDtypeStruct((Cl, L), src.dtype), mesh=mesh)
    def k(src_hbm, i_hbm, o_hbm):
        def body(i_vmem, o_vmem):
            pltpu.sync_copy(src_hbm.at[i_vmem.at[0]], o_vmem)

        pltpu.emit_pipeline(
            body,
            grid=(Cl // gw,),
            in_specs=[pl.BlockSpec((1, gw), index_map=lambda i: (0, i))],
            out_specs=[pl.BlockSpec((gw, L), index_map=lambda i: (i, 0))],
            core_axis_name=("c", "s"),
            dimension_semantics=(pltpu.PARALLEL,),
        )(i_hbm, o_hbm)

    return k(src, gidx)


def _writehead(th_l, Wt, b2, Wv, M, *, ndev):
    IN, Nl = th_l.shape
    C, L = M.shape
    N = Nl * ndev
    w_l, v_l, wr_l = _tc_call(th_l, Wt, b2, Wv)
    if ndev > 1:
        d = jax.lax.axis_index("x")
        wr = jnp.where(wr_l >= 0, wr_l + (d * Nl).astype(jnp.int32), -1)
        wrg = jax.lax.pmax(wr, "x")
        vfull = jax.lax.all_gather(v_l, "x", axis=0, tiled=True)
    else:
        d = 0
        wrg = wr_l
        vfull = v_l
    src = jnp.concatenate([vfull, M], axis=0)
    gidx = jnp.where(wrg >= 0, wrg, N + jnp.arange(C, dtype=jnp.int32)[None, :])
    Cl = C // ndev
    if ndev > 1:
        gidx_l = jax.lax.dynamic_slice(gidx, (0, d * Cl), (1, Cl))
    else:
        gidx_l = gidx
    M_new_l = _sc_gather(src, gidx_l, 128)
    return w_l, M_new_l


def kernel(thetas, W, b, M, Wv):
    C, L = M.shape
    devs = jax.devices()
    ndev = 2 if len(devs) >= 2 else 1
    Wt = W.T
    b2 = b.reshape(1, C)
    if ndev == 1:
        return _writehead(thetas, Wt, b2, Wv, M, ndev=1)
    mesh = Mesh(np.array(devs[:2]), ("x",))
    fn = shard_map(
        partial(_writehead, ndev=2),
        mesh=mesh,
        in_specs=(P(None, "x"), P(), P(), P(), P()),
        out_specs=(P("x", None), P("x", None)),
        check_rep=False,
    )
    return fn(thetas, Wt, b2, Wv, M)


# speedup vs baseline: 5.1980x; 5.1980x over previous
import jax
import jax.numpy as jnp
from jax.experimental import pallas as pl
from jax.experimental.pallas import tpu as pltpu
from jax.experimental.pallas import tpu_sc as plsc

_BN = 512
_NB = 16
_MB = 8
_GW = 128


def _tc_body(th_ref, wt_ref, b_ref, wv_ref, m_ref, w_ref, vext_ref, gidx_ref):
    i = pl.program_id(0)
    bn, C = w_ref.shape

    @pl.when(i < _NB)
    def _compute():
        th = th_ref[...]
        w = jax.lax.dot_general(th, wt_ref[...], (((0,), (0,)), ((), ())),
                                preferred_element_type=jnp.float32)
        w = w + b_ref[...]
        w_ref[...] = w
        vext_ref[...] = jax.lax.dot_general(
            th, wv_ref[...], (((0,), (0,)), ((), ())),
            preferred_element_type=jnp.float32)
        colio = jax.lax.broadcasted_iota(jnp.int32, (bn, C), 1)
        rowmax = jnp.max(w, axis=1, keepdims=True)
        idx = jnp.min(jnp.where(w == rowmax, colio, C), axis=1, keepdims=True)
        rowio = jax.lax.broadcasted_iota(jnp.int32, (bn, C), 0) + i * bn
        blkmax = jnp.max(jnp.where(colio == idx, rowio, -1), axis=0,
                         keepdims=True)

        @pl.when(i == 0)
        def _():
            gidx_ref[...] = blkmax

        @pl.when(i > 0)
        def _():
            gidx_ref[...] = jnp.maximum(gidx_ref[...], blkmax)

        @pl.when(i == _NB - 1)
        def _():
            wr = gidx_ref[...]
            cio = jax.lax.broadcasted_iota(jnp.int32, wr.shape, 1)
            gidx_ref[...] = jnp.where(wr >= 0, wr, _NB * bn + cio)

    @pl.when(i >= _NB)
    def _copy_m():
        vext_ref[...] = m_ref[...]


def _tc_call(thetas, Wt, b2, Wv, M):
    IN, N = thetas.shape
    C = Wt.shape[1]
    L = Wv.shape[1]
    grid = (_NB + _MB,)
    return pl.pallas_call(
        _tc_body,
        grid=grid,
        in_specs=[
            pl.BlockSpec((IN, _BN), lambda i: (0, jnp.minimum(i, _NB - 1))),
            pl.BlockSpec((IN, C), lambda i: (0, 0)),
            pl.BlockSpec((1, C), lambda i: (0, 0)),
            pl.BlockSpec((IN, L), lambda i: (0, 0)),
            pl.BlockSpec((_BN, L), lambda i: (jnp.maximum(i - _NB, 0), 0)),
        ],
        out_specs=[
            pl.BlockSpec((_BN, C), lambda i: (jnp.minimum(i, _NB - 1), 0)),
            pl.BlockSpec((_BN, L), lambda i: (i, 0)),
            pl.BlockSpec((1, C), lambda i: (0, 0)),
        ],
        out_shape=[
            jax.ShapeDtypeStruct((N, C), jnp.float32),
            jax.ShapeDtypeStruct((N + C, L), jnp.float32),
            jax.ShapeDtypeStruct((1, C), jnp.int32),
        ],
    )(thetas, Wt, b2, Wv, M)


def _sc_gather(src, gidx):
    C = gidx.shape[1]
    L = src.shape[1]
    mesh = plsc.VectorSubcoreMesh(core_axis_name="c", subcore_axis_name="s")

    @pl.kernel(out_type=jax.ShapeDtypeStruct((C, L), src.dtype), mesh=mesh)
    def k(src_hbm, i_hbm, o_hbm):
        def body(i_vmem, o_vmem):
            pltpu.sync_copy(src_hbm.at[i_vmem.at[0]], o_vmem)

        pltpu.emit_pipeline(
            body,
            grid=(C // _GW,),
            in_specs=[pl.BlockSpec((1, _GW), index_map=lambda i: (0, i))],
            out_specs=[pl.BlockSpec((_GW, L), index_map=lambda i: (i, 0))],
            core_axis_name=("c", "s"),
            dimension_semantics=(pltpu.PARALLEL,),
        )(i_hbm, o_hbm)

    return k(src, gidx)


def kernel(thetas, W, b, M, Wv):
    C, L = M.shape
    w, vext, gidx = _tc_call(thetas, W.T, b.reshape(1, C), Wv, M)
    M_new = _sc_gather(vext, gidx)
    return (w, M_new)


# speedup vs baseline: 5.3294x; 1.0253x over previous
import jax
import jax.numpy as jnp
from jax.experimental import pallas as pl
from jax.experimental.pallas import tpu as pltpu
from jax.experimental.pallas import tpu_sc as plsc

_BN = 512
_NB = 16
_MB = 8
_GW = 128


def _tc_body(th_ref, wt_ref, b_ref, wv_ref, m_ref, w_ref, vext_ref, gidx_ref):
    i = pl.program_id(0)
    bn, C = w_ref.shape

    @pl.when(i < _NB)
    def _compute():
        th = th_ref[...]
        w = jax.lax.dot_general(th, wt_ref[...], (((0,), (0,)), ((), ())),
                                preferred_element_type=jnp.float32)
        w = w + b_ref[...]
        w_ref[...] = w
        vext_ref[...] = jax.lax.dot_general(
            th, wv_ref[...], (((0,), (0,)), ((), ())),
            preferred_element_type=jnp.float32)
        rowmax = jnp.max(w, axis=1, keepdims=True)
        rowio_f = jax.lax.broadcasted_iota(
            jnp.int32, (bn, 1), 0).astype(jnp.float32)
        cand = jnp.where(w == rowmax, rowio_f, -1.0)
        blk_f = jnp.max(cand, axis=0, keepdims=True)
        tiesum = jnp.sum(cand)
        expected = float(bn * (bn - 1) // 2 - bn * (C - 1))
        noties = tiesum == expected

        def _update(blkmax):
            @pl.when(i == 0)
            def _():
                gidx_ref[...] = blkmax

            @pl.when(i > 0)
            def _():
                gidx_ref[...] = jnp.maximum(gidx_ref[...], blkmax)

        @pl.when(noties)
        def _fast():
            _update(jnp.where(blk_f >= 0.0,
                              blk_f.astype(jnp.int32) + i * bn, -1))

        @pl.when(jnp.logical_not(noties))
        def _exact():
            w2 = w_ref[...]
            rowmax2 = jnp.max(w2, axis=1, keepdims=True)
            colio = jax.lax.broadcasted_iota(jnp.int32, (bn, C), 1)
            idx = jnp.min(jnp.where(w2 == rowmax2, colio, C), axis=1,
                          keepdims=True)
            rowio = jax.lax.broadcasted_iota(jnp.int32, (bn, C), 0) + i * bn
            _update(jnp.max(jnp.where(colio == idx, rowio, -1), axis=0,
                            keepdims=True))

        @pl.when(i == _NB - 1)
        def _():
            wr = gidx_ref[...]
            cio = jax.lax.broadcasted_iota(jnp.int32, wr.shape, 1)
            gidx_ref[...] = jnp.where(wr >= 0, wr, _NB * bn + cio)

    @pl.when(i >= _NB)
    def _copy_m():
        vext_ref[...] = m_ref[...]


def _tc_call(thetas, Wt, b2, Wv, M):
    IN, N = thetas.shape
    C = Wt.shape[1]
    L = Wv.shape[1]
    grid = (_NB + _MB,)
    return pl.pallas_call(
        _tc_body,
        grid=grid,
        in_specs=[
            pl.BlockSpec((IN, _BN), lambda i: (0, jnp.minimum(i, _NB - 1))),
            pl.BlockSpec((IN, C), lambda i: (0, 0)),
            pl.BlockSpec((1, C), lambda i: (0, 0)),
            pl.BlockSpec((IN, L), lambda i: (0, 0)),
            pl.BlockSpec((_BN, L), lambda i: (jnp.maximum(i - _NB, 0), 0)),
        ],
        out_specs=[
            pl.BlockSpec((_BN, C), lambda i: (jnp.minimum(i, _NB - 1), 0)),
            pl.BlockSpec((_BN, L), lambda i: (i, 0)),
            pl.BlockSpec((1, C), lambda i: (0, 0)),
        ],
        out_shape=[
            jax.ShapeDtypeStruct((N, C), jnp.float32),
            jax.ShapeDtypeStruct((N + C, L), jnp.float32),
            jax.ShapeDtypeStruct((1, C), jnp.int32),
        ],
    )(thetas, Wt, b2, Wv, M)


def _sc_gather(src, gidx):
    C = gidx.shape[1]
    L = src.shape[1]
    mesh = plsc.VectorSubcoreMesh(core_axis_name="c", subcore_axis_name="s")

    @pl.kernel(out_type=jax.ShapeDtypeStruct((C, L), src.dtype), mesh=mesh)
    def k(src_hbm, i_hbm, o_hbm):
        def body(i_vmem, o_vmem):
            pltpu.sync_copy(src_hbm.at[i_vmem.at[0]], o_vmem)

        pltpu.emit_pipeline(
            body,
            grid=(C // _GW,),
            in_specs=[pl.BlockSpec((1, _GW), index_map=lambda i: (0, i))],
            out_specs=[pl.BlockSpec((_GW, L), index_map=lambda i: (i, 0))],
            core_axis_name=("c", "s"),
            dimension_semantics=(pltpu.PARALLEL,),
        )(i_hbm, o_hbm)

    return k(src, gidx)


def kernel(thetas, W, b, M, Wv):
    C, L = M.shape
    w, vext, gidx = _tc_call(thetas, W.T, b.reshape(1, C), Wv, M)
    M_new = _sc_gather(vext, gidx)
    return (w, M_new)
